# baseline (speedup 1.0000x reference)
import functools

import jax
import jax.numpy as jnp
from jax.experimental import pallas as pl
from jax.experimental.pallas import tpu as pltpu

N_OUT = 255
N_TILES = 8


def _cde_body(x_hbm, w_ref, b_ref, y_hbm, xbuf, ybuf, in_sem, out_sem):
    tile = xbuf.shape[0]

    def step(i, _):
        cin = pltpu.make_async_copy(
            x_hbm.at[pl.ds(i * tile, tile), :], xbuf, in_sem)
        cin.start()
        cin.wait()
        acc = jnp.dot(xbuf[...], w_ref[...],
                      preferred_element_type=jnp.float32)
        ybuf[...] = (acc + b_ref[...])[:, :N_OUT]
        cout = pltpu.make_async_copy(
            ybuf, y_hbm.at[pl.ds(i * tile, tile), :], out_sem)
        cout.start()
        cout.wait()
        return 0

    jax.lax.fori_loop(0, N_TILES, step, 0)


@jax.jit
def _forward(x, w_t_pad, b_pad):
    B, d_in = x.shape
    tile = B // N_TILES
    return pl.pallas_call(
        _cde_body,
        out_shape=jax.ShapeDtypeStruct((B, N_OUT), x.dtype),
        in_specs=[
            pl.BlockSpec(memory_space=pl.ANY),
            pl.BlockSpec(memory_space=pltpu.VMEM),
            pl.BlockSpec(memory_space=pltpu.VMEM),
        ],
        out_specs=pl.BlockSpec(memory_space=pl.ANY),
        scratch_shapes=[
            pltpu.VMEM((tile, d_in), jnp.float32),
            pltpu.VMEM((tile, N_OUT), jnp.float32),
            pltpu.SemaphoreType.DMA,
            pltpu.SemaphoreType.DMA,
        ],
    )(x, w_t_pad, b_pad)


def kernel(x, w_t_pad, b_pad):
    return _forward(x, w_t_pad, b_pad)


# speedup vs baseline: 1.3647x; 1.3647x over previous
import functools

import jax
import jax.numpy as jnp
from jax.experimental import pallas as pl
from jax.experimental.pallas import tpu as pltpu

N_OUT = 255
TILE_M = 13312


def _cde_kernel(x_ref, w_ref, b_ref, o_ref):
    acc = jnp.dot(x_ref[...], w_ref[...], preferred_element_type=jnp.float32)
    o_ref[...] = (acc + b_ref[...])[:, : o_ref.shape[-1]].astype(o_ref.dtype)


@jax.jit
def _forward(x, w_t_pad, b_pad):
    B, d_in = x.shape
    n_pad = w_t_pad.shape[1]
    tm = min(TILE_M, B)
    grid = (pl.cdiv(B, tm),)
    return pl.pallas_call(
        _cde_kernel,
        out_shape=jax.ShapeDtypeStruct((B, N_OUT), x.dtype),
        grid=grid,
        in_specs=[
            pl.BlockSpec((tm, d_in), lambda i: (i, 0)),
            pl.BlockSpec((d_in, n_pad), lambda i: (0, 0)),
            pl.BlockSpec((1, n_pad), lambda i: (0, 0)),
        ],
        out_specs=pl.BlockSpec((tm, N_OUT), lambda i: (i, 0)),
        compiler_params=pltpu.CompilerParams(
            dimension_semantics=("parallel",),
        ),
    )(x, w_t_pad, b_pad)


def kernel(x, w_t_pad, b_pad):
    return _forward(x, w_t_pad, b_pad)
